# baseline (speedup 1.0000x reference)
import functools

import jax
import jax.numpy as jnp
from jax import lax
from jax.experimental import pallas as pl
from jax.experimental.pallas import tpu as pltpu
from jax.experimental.pallas import tpu_sc as plsc

_TOKENS = 32
_VOCAB = 1_000_000
_EPS = 1e-10

_NC = 2
_NS = 16
_L = 16

_CHUNK = 8192
_FULL = _VOCAB // _CHUNK
_TAIL = _VOCAB - _FULL * _CHUNK
_UNROLL = 8


def _noise_table_body(exp_ref, out_ref):
    nlog = jnp.log(exp_ref[...] + _EPS)
    out_ref[...] = jnp.concatenate([nlog, jnp.zeros_like(nlog)], axis=0)


def _make_noise_table(exponential):
    blk = 131072
    grid = (_VOCAB + blk - 1) // blk
    return pl.pallas_call(
        _noise_table_body,
        grid=(grid,),
        in_specs=[pl.BlockSpec((1, blk), lambda i: (0, i))],
        out_specs=pl.BlockSpec((2, blk), lambda i: (0, i)),
        out_shape=jax.ShapeDtypeStruct((2, _VOCAB), jnp.float32),
    )(exponential)


def _scan_vectors(lbuf, nbuf, tvec, base, nvec, mval, midx):
    iv0 = jnp.full((_L,), base, jnp.int32) + lax.iota(jnp.int32, _L)

    def body(j, carry):
        mval, midx, iv = carry
        for u in range(_UNROLL):
            off = (j * _UNROLL + u) * _L
            s = lbuf[pl.ds(off, _L)] * tvec - nbuf[pl.ds(off, _L)]
            upd = s > mval
            mval = jnp.where(upd, s, mval)
            midx = jnp.where(upd, iv, midx)
            iv = iv + _L
        return mval, midx, iv

    mval, midx, _ = lax.fori_loop(0, nvec // _UNROLL, body, (mval, midx, iv0))
    for k in range(nvec - nvec % _UNROLL, nvec):
        off = k * _L
        s = lbuf[pl.ds(off, _L)] * tvec - nbuf[pl.ds(off, _L)]
        upd = s > mval
        mval = jnp.where(upd, s, mval)
        midx = jnp.where(upd, iv0 + off, midx)
    return mval, midx


def _sampler_body(logits, invt, rsel, ntab, out,
                  lbufA, lbufB, nbufA, nbufB, ltail, ntail,
                  scal_v, outbuf,
                  semLA, semLB, semNA, semNB, semLT, semNT):
    t = lax.axis_index("s") * _NC + lax.axis_index("c")

    pltpu.sync_copy(invt, scal_v.at[pl.ds(0, _TOKENS)])
    tfull = jnp.full((_L,), t, jnp.int32)
    tvec = plsc.load_gather(scal_v.at[pl.ds(0, _TOKENS)], [tfull])
    pltpu.sync_copy(rsel, scal_v.at[pl.ds(_TOKENS, _TOKENS)])
    rvec = plsc.load_gather(scal_v.at[pl.ds(_TOKENS, _TOKENS)], [tfull])
    r = jnp.max(rvec.astype(jnp.int32))

    def start(c, lbuf, nbuf, semL, semN):
        pltpu.async_copy(logits.at[t, pl.ds(c * _CHUNK, _CHUNK)], lbuf, semL)
        pltpu.async_copy(ntab.at[r, pl.ds(c * _CHUNK, _CHUNK)], nbuf, semN)

    def wait(lbuf, nbuf, semL, semN):
        pltpu.make_async_copy(
            logits.at[t, pl.ds(0, _CHUNK)], lbuf, semL).wait()
        pltpu.make_async_copy(
            ntab.at[0, pl.ds(0, _CHUNK)], nbuf, semN).wait()

    start(0, lbufA, nbufA, semLA, semNA)
    start(1, lbufB, nbufB, semLB, semNB)
    pltpu.async_copy(
        logits.at[t, pl.ds(_FULL * _CHUNK, _TAIL)], ltail, semLT)
    pltpu.async_copy(
        ntab.at[r, pl.ds(_FULL * _CHUNK, _TAIL)], ntail, semNT)

    mval0 = jnp.full((_L,), -jnp.inf, jnp.float32)
    midx0 = jnp.zeros((_L,), jnp.int32)

    def pair(i, carry):
        mval, midx = carry
        c0 = 2 * i
        wait(lbufA, nbufA, semLA, semNA)
        mval, midx = _scan_vectors(lbufA, nbufA, tvec, c0 * _CHUNK,
                                   _CHUNK // _L, mval, midx)

        @pl.when(c0 + 2 < _FULL)
        def _():
            start(c0 + 2, lbufA, nbufA, semLA, semNA)

        wait(lbufB, nbufB, semLB, semNB)
        mval, midx = _scan_vectors(lbufB, nbufB, tvec, (c0 + 1) * _CHUNK,
                                   _CHUNK // _L, mval, midx)

        @pl.when(c0 + 3 < _FULL)
        def _():
            start(c0 + 3, lbufB, nbufB, semLB, semNB)

        return mval, midx

    mval, midx = lax.fori_loop(0, _FULL // 2, pair, (mval0, midx0))

    pltpu.make_async_copy(
        logits.at[t, pl.ds(0, _TAIL)], ltail, semLT).wait()
    pltpu.make_async_copy(
        ntab.at[0, pl.ds(0, _TAIL)], ntail, semNT).wait()
    mval, midx = _scan_vectors(ltail, ntail, tvec, _FULL * _CHUNK,
                               _TAIL // _L, mval, midx)

    best = jnp.max(mval)
    cand = jnp.where(mval == best, midx, jnp.int32(2**31 - 1))
    token = jnp.min(cand)
    outbuf[...] = jnp.full((_L,), token, jnp.int32)
    pltpu.sync_copy(outbuf, out.at[t])


_sampler = functools.partial(
    pl.kernel,
    out_type=jax.ShapeDtypeStruct((_TOKENS, _L), jnp.int32),
    mesh=plsc.VectorSubcoreMesh(
        core_axis_name="c", subcore_axis_name="s",
        num_cores=_NC, num_subcores=_NS),
    compiler_params=pltpu.CompilerParams(
        use_tc_tiling_on_sc=False, needs_layout_passes=False),
    scratch_types=[
        pltpu.VMEM((_CHUNK,), jnp.float32),
        pltpu.VMEM((_CHUNK,), jnp.float32),
        pltpu.VMEM((_CHUNK,), jnp.float32),
        pltpu.VMEM((_CHUNK,), jnp.float32),
        pltpu.VMEM((_TAIL,), jnp.float32),
        pltpu.VMEM((_TAIL,), jnp.float32),
        pltpu.VMEM((2 * _TOKENS,), jnp.float32),
        pltpu.VMEM((_L,), jnp.int32),
        pltpu.SemaphoreType.DMA,
        pltpu.SemaphoreType.DMA,
        pltpu.SemaphoreType.DMA,
        pltpu.SemaphoreType.DMA,
        pltpu.SemaphoreType.DMA,
        pltpu.SemaphoreType.DMA,
    ],
)(_sampler_body)


@jax.jit
def kernel(logits, temperatures, exponential):
    ntab = _make_noise_table(exponential)
    pos = temperatures > 0
    invt = jnp.where(pos, 1.0 / jnp.where(pos, temperatures, 1.0), 1.0)
    rsel = jnp.where(pos, 0, 1).astype(jnp.float32)
    res = _sampler(logits, invt, rsel, ntab)
    return res[:, 0]
